# Initial kernel scaffold; baseline (speedup 1.0000x reference)
import jax
import jax.numpy as jnp
from jax.experimental import pallas as pl


def kernel(h, u, edge_index, W_fc, W_attn2):
    raise NotImplementedError("write your pallas kernel here")


# speedup vs baseline: 124.9264x; 124.9264x over previous
import functools

import jax
import jax.numpy as jnp
from jax import lax
from jax.experimental import pallas as pl
from jax.experimental.pallas import tpu as pltpu
from jax.experimental.pallas import tpu_sc as plsc

N = 10000
D = 128
E = 320000
NC = 1
NS = 16
NW = NC * NS
EDGES_PER_TILE = E // NW
CHUNK = 80
ECHUNKS = EDGES_PER_TILE // CHUNK
ROWS_PER_TILE = 624
ZROWS = 16
DEGW = 16


def _matmul_body(h_ref, w_ref, z_ref):
    z_ref[...] = lax.dot_general(
        h_ref[...], w_ref[...],
        dimension_numbers=(((1,), (1,)), ((), ())),
        preferred_element_type=jnp.float32,
        precision=lax.Precision.HIGHEST,
    )


def _matmul(h, w):
    blk = 1000
    return pl.pallas_call(
        _matmul_body,
        grid=(N // blk,),
        in_specs=[
            pl.BlockSpec((blk, D), lambda i: (i, 0)),
            pl.BlockSpec((D, D), lambda i: (0, 0)),
        ],
        out_specs=pl.BlockSpec((blk, D), lambda i: (i, 0)),
        out_shape=jax.ShapeDtypeStruct((N, D), jnp.float32),
    )(h, w)


def _agg_body(z_hbm, src_hbm, dst_hbm, acc_hbm, deg_hbm,
              srcv, dstv, rows, ones, zbuf, dzbuf, acc_sh, deg_sh, sem):
    cid = lax.axis_index("c")
    sid = lax.axis_index("s")
    wid = cid * NS + sid

    @pl.loop(0, CHUNK)
    def _(i):
        ones[i, :] = jnp.full((DEGW,), 1.0, jnp.float32)

    @pl.loop(0, ZROWS)
    def _(i):
        dzbuf[i, :] = jnp.zeros((DEGW,), jnp.float32)

        @pl.loop(0, D, step=16)
        def _(j):
            zbuf[i, pl.ds(j, 16)] = jnp.zeros((16,), jnp.float32)

    r0 = sid * ROWS_PER_TILE

    @pl.loop(0, ROWS_PER_TILE // ZROWS)
    def _(i):
        pltpu.sync_copy(zbuf, acc_sh.at[pl.ds(r0 + i * ZROWS, ZROWS)])
        pltpu.sync_copy(dzbuf, deg_sh.at[pl.ds(r0 + i * ZROWS, ZROWS)])

    @pl.when(sid == NS - 1)
    def _():
        tail = NS * ROWS_PER_TILE
        pltpu.sync_copy(zbuf.at[pl.ds(0, N - tail)], acc_sh.at[pl.ds(tail, N - tail)])
        pltpu.sync_copy(dzbuf.at[pl.ds(0, N - tail)], deg_sh.at[pl.ds(tail, N - tail)])

    plsc.subcore_barrier()

    ebase = wid * EDGES_PER_TILE

    @pl.loop(0, ECHUNKS)
    def _(j):
        pltpu.sync_copy(src_hbm.at[pl.ds(ebase + j * CHUNK, CHUNK)], srcv)
        pltpu.sync_copy(dst_hbm.at[pl.ds(ebase + j * CHUNK, CHUNK)], dstv)
        pltpu.async_copy(z_hbm.at[srcv], rows, sem).wait()
        pltpu.sync_copy(rows, acc_sh.at[dstv], add=True)
        pltpu.sync_copy(ones, deg_sh.at[dstv], add=True)

    plsc.subcore_barrier()

    obase = cid * N + r0
    pltpu.sync_copy(acc_sh.at[pl.ds(r0, ROWS_PER_TILE)],
                    acc_hbm.at[pl.ds(obase, ROWS_PER_TILE)])
    pltpu.sync_copy(deg_sh.at[pl.ds(r0, ROWS_PER_TILE)],
                    deg_hbm.at[pl.ds(obase, ROWS_PER_TILE)])

    @pl.when(sid == NS - 1)
    def _():
        tail = NS * ROWS_PER_TILE
        pltpu.sync_copy(acc_sh.at[pl.ds(tail, N - tail)],
                        acc_hbm.at[pl.ds(cid * N + tail, N - tail)])
        pltpu.sync_copy(deg_sh.at[pl.ds(tail, N - tail)],
                        deg_hbm.at[pl.ds(cid * N + tail, N - tail)])


def _aggregate(z, src2, dst2):
    mesh = plsc.VectorSubcoreMesh(core_axis_name="c", subcore_axis_name="s",
                                  num_cores=NC)
    agg = functools.partial(
        pl.kernel,
        out_type=(
            jax.ShapeDtypeStruct((NC * N, D), jnp.float32),
            jax.ShapeDtypeStruct((NC * N, DEGW), jnp.float32),
        ),
        mesh=mesh,
        scratch_types=[
            pltpu.VMEM((CHUNK,), jnp.int32),
            pltpu.VMEM((CHUNK,), jnp.int32),
            pltpu.VMEM((CHUNK, D), jnp.float32),
            pltpu.VMEM((CHUNK, DEGW), jnp.float32),
            pltpu.VMEM((ZROWS, D), jnp.float32),
            pltpu.VMEM((ZROWS, DEGW), jnp.float32),
            pltpu.VMEM_SHARED((N, D), jnp.float32),
            pltpu.VMEM_SHARED((N, DEGW), jnp.float32),
            pltpu.SemaphoreType.DMA,
        ],
        compiler_params=pltpu.CompilerParams(use_tc_tiling_on_sc=False),
    )(_agg_body)
    return agg(z, src2, dst2)


def _finalize_body(acc_ref, deg_ref, out_ref):
    a = acc_ref[0]
    d = deg_ref[0, :, 0:1]
    for c in range(1, NC):
        a = a + acc_ref[c]
        d = d + deg_ref[c, :, 0:1]
    inv = jnp.where(d > 0.0, 1.0 / jnp.maximum(d, 1.0), 0.0)
    out_ref[...] = a * inv


def _finalize(acc, deg):
    blk = 1000
    return pl.pallas_call(
        _finalize_body,
        grid=(N // blk,),
        in_specs=[
            pl.BlockSpec((NC, blk, D), lambda i: (0, i, 0)),
            pl.BlockSpec((NC, blk, DEGW), lambda i: (0, i, 0)),
        ],
        out_specs=pl.BlockSpec((blk, D), lambda i: (i, 0)),
        out_shape=jax.ShapeDtypeStruct((N, D), jnp.float32),
    )(acc, deg)


def kernel(h, u, edge_index, W_fc, W_attn2):
    z = _matmul(h, W_fc)
    src2 = edge_index[0]
    dst2 = edge_index[1]
    acc, deg = _aggregate(z, src2, dst2)
    return _finalize(acc.reshape(NC, N, D), deg.reshape(NC, N, DEGW))


# speedup vs baseline: 230.0814x; 1.8417x over previous
import functools

import jax
import jax.numpy as jnp
from jax import lax
from jax.experimental import pallas as pl
from jax.experimental.pallas import tpu as pltpu
from jax.experimental.pallas import tpu_sc as plsc

N = 10000
D = 128
E = 320000
NC = 1
NS = 16
NW = NC * NS
EDGES_PER_TILE = E // NW
CHUNK = 80
ECHUNKS = EDGES_PER_TILE // CHUNK
SUPER = 50
NSUPER = ECHUNKS // SUPER
ROWS_PER_TILE = 624
ZROWS = 16
DEGW = 16


def _matmul_body(h_ref, w_ref, z_ref):
    z_ref[...] = lax.dot_general(
        h_ref[...], w_ref[...],
        dimension_numbers=(((1,), (1,)), ((), ())),
        preferred_element_type=jnp.float32,
        precision=lax.Precision.HIGHEST,
    )


def _matmul(h, w):
    blk = 1000
    return pl.pallas_call(
        _matmul_body,
        grid=(N // blk,),
        in_specs=[
            pl.BlockSpec((blk, D), lambda i: (i, 0)),
            pl.BlockSpec((D, D), lambda i: (0, 0)),
        ],
        out_specs=pl.BlockSpec((blk, D), lambda i: (i, 0)),
        out_shape=jax.ShapeDtypeStruct((N, D), jnp.float32),
    )(h, w)


def _agg_body(z_hbm, src_hbm, dst_hbm, acc_hbm, deg_hbm,
              sidx, didx, rows0, rows1, ones, zbuf, dzbuf, acc_sh, deg_sh,
              gsem0, gsem1, ssem0, ssem1, osem0, osem1):
    cid = lax.axis_index("c")
    sid = lax.axis_index("s")
    wid = cid * NS + sid

    @pl.loop(0, CHUNK)
    def _(i):
        ones[i, :] = jnp.full((DEGW,), 1.0, jnp.float32)

    @pl.loop(0, ZROWS)
    def _(i):
        dzbuf[i, :] = jnp.zeros((DEGW,), jnp.float32)

        @pl.loop(0, D, step=16)
        def _(j):
            zbuf[i, pl.ds(j, 16)] = jnp.zeros((16,), jnp.float32)

    r0 = sid * ROWS_PER_TILE

    @pl.loop(0, ROWS_PER_TILE // ZROWS)
    def _(i):
        pltpu.sync_copy(zbuf, acc_sh.at[pl.ds(r0 + i * ZROWS, ZROWS)])
        pltpu.sync_copy(dzbuf, deg_sh.at[pl.ds(r0 + i * ZROWS, ZROWS)])

    @pl.when(sid == NS - 1)
    def _():
        tail = NS * ROWS_PER_TILE
        pltpu.sync_copy(zbuf.at[pl.ds(0, N - tail)], acc_sh.at[pl.ds(tail, N - tail)])
        pltpu.sync_copy(dzbuf.at[pl.ds(0, N - tail)], deg_sh.at[pl.ds(tail, N - tail)])

    plsc.subcore_barrier()

    cbase = wid * ECHUNKS

    @pl.loop(0, NSUPER)
    def _(g):
        pltpu.sync_copy(src_hbm.at[pl.ds(cbase + g * SUPER, SUPER)], sidx)
        pltpu.sync_copy(dst_hbm.at[pl.ds(cbase + g * SUPER, SUPER)], didx)

        @pl.loop(0, SUPER, step=2)
        def _(m):
            c0 = pltpu.async_copy(z_hbm.at[sidx.at[m]], rows0, gsem0)
            c1 = pltpu.async_copy(z_hbm.at[sidx.at[m + 1]], rows1, gsem1)
            c0.wait()
            s0 = pltpu.async_copy(rows0, acc_sh.at[didx.at[m]], ssem0, add=True)
            o0 = pltpu.async_copy(ones, deg_sh.at[didx.at[m]], osem0, add=True)
            c1.wait()
            s1 = pltpu.async_copy(rows1, acc_sh.at[didx.at[m + 1]], ssem1,
                                  add=True)
            o1 = pltpu.async_copy(ones, deg_sh.at[didx.at[m + 1]], osem1,
                                  add=True)
            s0.wait()
            o0.wait()
            s1.wait()
            o1.wait()

    plsc.subcore_barrier()

    obase = cid * N + r0
    pltpu.sync_copy(acc_sh.at[pl.ds(r0, ROWS_PER_TILE)],
                    acc_hbm.at[pl.ds(obase, ROWS_PER_TILE)])
    pltpu.sync_copy(deg_sh.at[pl.ds(r0, ROWS_PER_TILE)],
                    deg_hbm.at[pl.ds(obase, ROWS_PER_TILE)])

    @pl.when(sid == NS - 1)
    def _():
        tail = NS * ROWS_PER_TILE
        pltpu.sync_copy(acc_sh.at[pl.ds(tail, N - tail)],
                        acc_hbm.at[pl.ds(cid * N + tail, N - tail)])
        pltpu.sync_copy(deg_sh.at[pl.ds(tail, N - tail)],
                        deg_hbm.at[pl.ds(cid * N + tail, N - tail)])


def _aggregate(z, src2, dst2):
    mesh = plsc.VectorSubcoreMesh(core_axis_name="c", subcore_axis_name="s",
                                  num_cores=NC)
    agg = functools.partial(
        pl.kernel,
        out_type=(
            jax.ShapeDtypeStruct((NC * N, D), jnp.float32),
            jax.ShapeDtypeStruct((NC * N, DEGW), jnp.float32),
        ),
        mesh=mesh,
        scratch_types=[
            pltpu.VMEM((SUPER, CHUNK), jnp.int32),
            pltpu.VMEM((SUPER, CHUNK), jnp.int32),
            pltpu.VMEM((CHUNK, D), jnp.float32),
            pltpu.VMEM((CHUNK, D), jnp.float32),
            pltpu.VMEM((CHUNK, DEGW), jnp.float32),
            pltpu.VMEM((ZROWS, D), jnp.float32),
            pltpu.VMEM((ZROWS, DEGW), jnp.float32),
            pltpu.VMEM_SHARED((N, D), jnp.float32),
            pltpu.VMEM_SHARED((N, DEGW), jnp.float32),
            pltpu.SemaphoreType.DMA,
            pltpu.SemaphoreType.DMA,
            pltpu.SemaphoreType.DMA,
            pltpu.SemaphoreType.DMA,
            pltpu.SemaphoreType.DMA,
            pltpu.SemaphoreType.DMA,
        ],
        compiler_params=pltpu.CompilerParams(use_tc_tiling_on_sc=False),
    )(_agg_body)
    return agg(z, src2, dst2)


def _finalize_body(acc_ref, deg_ref, out_ref):
    a = acc_ref[0]
    d = deg_ref[0, :, 0:1]
    for c in range(1, NC):
        a = a + acc_ref[c]
        d = d + deg_ref[c, :, 0:1]
    inv = jnp.where(d > 0.0, 1.0 / jnp.maximum(d, 1.0), 0.0)
    out_ref[...] = a * inv


def _finalize(acc, deg):
    blk = 1000
    return pl.pallas_call(
        _finalize_body,
        grid=(N // blk,),
        in_specs=[
            pl.BlockSpec((NC, blk, D), lambda i: (0, i, 0)),
            pl.BlockSpec((NC, blk, DEGW), lambda i: (0, i, 0)),
        ],
        out_specs=pl.BlockSpec((blk, D), lambda i: (i, 0)),
        out_shape=jax.ShapeDtypeStruct((N, D), jnp.float32),
    )(acc, deg)


def kernel(h, u, edge_index, W_fc, W_attn2):
    z = _matmul(h, W_fc)
    src2 = edge_index[0].reshape(NW * ECHUNKS, CHUNK)
    dst2 = edge_index[1].reshape(NW * ECHUNKS, CHUNK)
    acc, deg = _aggregate(z, src2, dst2)
    return _finalize(acc.reshape(NC, N, D), deg.reshape(NC, N, DEGW))
